# baseline (speedup 1.0000x reference)
import functools

import jax
import jax.numpy as jnp
from jax import lax
from jax.experimental import pallas as pl
from jax.experimental.pallas import tpu as pltpu
from jax.experimental.pallas import tpu_sc as plsc

F = 26
V = 100000
D = 32
B = 16384

NW = 32
TT = F * D
RPT = TT // NW
NCK = 4
CB = B // NCK
L = 16


@functools.partial(
    pl.kernel,
    out_type=jax.ShapeDtypeStruct((TT, B), jnp.float32),
    mesh=plsc.VectorSubcoreMesh(core_axis_name="c", subcore_axis_name="s"),
    scratch_types=(
        [pltpu.VMEM((V,), jnp.float32),
         pltpu.VMEM((B,), jnp.int32),
         pltpu.VMEM((2, CB), jnp.float32)]
        + [pltpu.SemaphoreType.DMA] * 5
    ),
    compiler_params=pltpu.CompilerParams(needs_layout_passes=False),
)
def _embed_rows(xt_hbm, tabt_hbm, out_hbm, row_v, idx_v, val_v,
                rsem, xsem0, xsem1, vsem0, vsem1):
    xsems = (xsem0, xsem1)
    vsems = (vsem0, vsem1)
    w = lax.axis_index("s") * 2 + lax.axis_index("c")
    grp = w // 8
    j = w - grp * 8

    def row_body(k, prev_stores):
        o = grp * RPT + k
        f = o // 4
        g = o - f * 4
        d = g * 8 + j
        r = f * D + d
        h_row = pltpu.async_copy(tabt_hbm.at[f, d], row_v, rsem)
        h_x = pltpu.async_copy(xt_hbm.at[f], idx_v, xsems[0])
        h_v = [None] * NCK
        for s in prev_stores:
            s.wait()
        h_x.wait()
        h_row.wait()
        for c in range(NCK):
            if c >= 2:
                h_v[c - 2].wait()
            p = c % 2

            @plsc.parallel_loop(0, CB, step=L, unroll=8)
            def gbody(i, c=c, p=p):
                sl = pl.ds(c * CB + i, L)
                val_v[p, pl.ds(i, L)] = plsc.load_gather(row_v, [idx_v[sl]])

            h_v[c] = pltpu.async_copy(
                val_v.at[p], out_hbm.at[r, pl.ds(c * CB, CB)], vsems[p])
        return [h_v[NCK - 2], h_v[NCK - 1]]

    stores = []
    for k in range(RPT):
        stores = row_body(k, stores)
    for s in stores:
        s.wait()


def kernel(x, tables):
    xt = x.T
    tabt = jnp.transpose(tables, (0, 2, 1))
    out = _embed_rows(xt, tabt)
    return out.T


# speedup vs baseline: 1.0357x; 1.0357x over previous
import functools

import jax
import jax.numpy as jnp
from jax import lax
from jax.experimental import pallas as pl
from jax.experimental.pallas import tpu as pltpu
from jax.experimental.pallas import tpu_sc as plsc

F = 26
V = 100000
D = 32
B = 16384

NW = 32
TT = F * D
RPT = TT // NW
NCK = 4
CB = B // NCK
L = 16


@functools.partial(
    pl.kernel,
    out_type=jax.ShapeDtypeStruct((TT, B), jnp.float32),
    mesh=plsc.VectorSubcoreMesh(core_axis_name="c", subcore_axis_name="s"),
    scratch_types=(
        [pltpu.VMEM((V,), jnp.float32),
         pltpu.VMEM((2, CB), jnp.int32),
         pltpu.VMEM((2, CB), jnp.float32)]
        + [pltpu.SemaphoreType.DMA] * 5
    ),
    compiler_params=pltpu.CompilerParams(needs_layout_passes=False),
)
def _embed_rows(xt_hbm, tabt_hbm, out_hbm, row_v, idx_v, val_v,
                rsem, xsem0, xsem1, vsem0, vsem1):
    xsems = (xsem0, xsem1)
    vsems = (vsem0, vsem1)
    w = lax.axis_index("s") * 2 + lax.axis_index("c")
    grp = w // 8
    j = w - grp * 8

    def row_body(k, prev_stores):
        o = grp * RPT + k
        f = o // 4
        g = o - f * 4
        d = g * 8 + j
        r = f * D + d
        h_row = pltpu.async_copy(tabt_hbm.at[f, d], row_v, rsem)
        h_x = [None] * NCK
        h_x[0] = pltpu.async_copy(
            xt_hbm.at[f, pl.ds(0, CB)], idx_v.at[0], xsems[0])
        h_v = [None] * NCK
        for s in prev_stores:
            s.wait()
        h_row.wait()
        for c in range(NCK):
            if c + 1 < NCK:
                h_x[c + 1] = pltpu.async_copy(
                    xt_hbm.at[f, pl.ds((c + 1) * CB, CB)],
                    idx_v.at[(c + 1) % 2], xsems[(c + 1) % 2])
            h_x[c].wait()
            if c >= 2:
                h_v[c - 2].wait()
            p = c % 2

            @plsc.parallel_loop(0, CB, step=L, unroll=8)
            def gbody(i, p=p):
                sl = pl.ds(i, L)
                val_v[p, sl] = plsc.load_gather(row_v, [idx_v[p, sl]])

            h_v[c] = pltpu.async_copy(
                val_v.at[p], out_hbm.at[r, pl.ds(c * CB, CB)], vsems[p])
        return [h_v[NCK - 2], h_v[NCK - 1]]

    stores = []
    for k in range(RPT):
        stores = row_body(k, stores)
    for s in stores:
        s.wait()


def kernel(x, tables):
    xt = x.T
    tabt = jnp.transpose(tables, (0, 2, 1))
    out = _embed_rows(xt, tabt)
    return out.T
